# Initial kernel scaffold; baseline (speedup 1.0000x reference)
import jax
import jax.numpy as jnp
from jax.experimental import pallas as pl


def kernel(x, angle_geom, ca_coords, edge_index, batch, W_emb, b_emb, W_rbf, W_sbf, W_l0, W_l1, W_l2, b_l0, b_l1, b_l2, W_out, W_score, b_score, W_cls, b_cls):
    raise NotImplementedError("write your pallas kernel here")


# speedup vs baseline: 1.0812x; 1.0812x over previous
import functools
import math

import jax
import jax.numpy as jnp
from jax import lax
from jax.experimental import pallas as pl
from jax.experimental.pallas import tpu as pltpu
from jax.experimental.pallas import tpu_sc as plsc

N = 10000
E = 320000
H = 128
HID4 = 512
B = 16
K = 2
OH = 23
NR = 6
NS = 7
CUT = 5.0
PRED = 8

NC = 2
NSUB = 16
NW = NC * NSUB
PER_W = E // NW
CH = 80
NCHUNK = PER_W // CH
RPT = 624
RTAIL = N - RPT * NSUB

_SC_MESH = plsc.VectorSubcoreMesh(core_axis_name="c", subcore_axis_name="s")
_SC_PARAMS = pltpu.CompilerParams(needs_layout_passes=False)

NP = 10240
NROW = NP // 128


@functools.partial(
    pl.kernel,
    mesh=_SC_MESH,
    compiler_params=_SC_PARAMS,
    out_type=jax.ShapeDtypeStruct((E,), jnp.float32),
    scratch_types=[
        pltpu.VMEM((N,), jnp.float32),
        pltpu.VMEM((N,), jnp.float32),
        pltpu.VMEM((N,), jnp.float32),
        pltpu.VMEM((CH,), jnp.int32),
        pltpu.VMEM((CH,), jnp.int32),
        pltpu.VMEM((CH,), jnp.float32),
    ],
)
def _dist_sc(cax_h, cay_h, caz_h, src_h, dst_h, out_h,
             cax_v, cay_v, caz_v, sidx_v, didx_v, dd_v):
    wid = lax.axis_index("s") * NC + lax.axis_index("c")
    base = wid * PER_W
    pltpu.sync_copy(cax_h, cax_v)
    pltpu.sync_copy(cay_h, cay_v)
    pltpu.sync_copy(caz_h, caz_v)

    def chunk(i, carry):
        off = base + i * CH
        pltpu.sync_copy(src_h.at[pl.ds(off, CH)], sidx_v)
        pltpu.sync_copy(dst_h.at[pl.ds(off, CH)], didx_v)
        for j in range(CH // 16):
            si = sidx_v[pl.ds(j * 16, 16)]
            di = didx_v[pl.ds(j * 16, 16)]
            ddx = plsc.load_gather(cax_v, [di]) - plsc.load_gather(cax_v, [si])
            ddy = plsc.load_gather(cay_v, [di]) - plsc.load_gather(cay_v, [si])
            ddz = plsc.load_gather(caz_v, [di]) - plsc.load_gather(caz_v, [si])
            dd_v[pl.ds(j * 16, 16)] = ddx * ddx + ddy * ddy + ddz * ddz
        pltpu.sync_copy(dd_v, out_h.at[pl.ds(off, CH)])
        return carry

    lax.fori_loop(0, NCHUNK, chunk, 0)


@functools.partial(
    pl.kernel,
    mesh=_SC_MESH,
    out_type=jax.ShapeDtypeStruct((E, H), jnp.float32),
    scratch_types=[
        pltpu.VMEM((CH,), jnp.int32),
        pltpu.VMEM((CH, H), jnp.float32),
        pltpu.SemaphoreType.DMA,
    ],
)
def _gather_sc(h_h, src_h, out_h, idx_v, rows_v, sem):
    wid = lax.axis_index("s") * NC + lax.axis_index("c")
    base = wid * PER_W

    def chunk(i, carry):
        off = base + i * CH
        pltpu.sync_copy(src_h.at[pl.ds(off, CH)], idx_v)
        pltpu.async_copy(h_h.at[idx_v], rows_v, sem).wait()
        pltpu.sync_copy(rows_v, out_h.at[pl.ds(off, CH)])
        return carry

    lax.fori_loop(0, NCHUNK, chunk, 0)


@functools.partial(
    pl.kernel,
    mesh=_SC_MESH,
    out_type=jax.ShapeDtypeStruct((NC * N, H), jnp.float32),
    scratch_types=[
        pltpu.VMEM((CH,), jnp.int32),
        pltpu.VMEM((CH, H), jnp.float32),
        pltpu.VMEM_SHARED((N, H), jnp.float32),
    ],
)
def _scatter_sc(m_h, dst_h, zeros_h, out_h, idx_v, rows_v, acc_sh):
    cid = lax.axis_index("c")
    sid = lax.axis_index("s")
    wid = sid * NC + cid
    base = wid * PER_W
    pltpu.sync_copy(zeros_h.at[pl.ds(sid * RPT, RPT)],
                    acc_sh.at[pl.ds(sid * RPT, RPT)])

    @pl.when(sid == NSUB - 1)
    def _():
        pltpu.sync_copy(zeros_h.at[pl.ds(RPT * NSUB, RTAIL)],
                        acc_sh.at[pl.ds(RPT * NSUB, RTAIL)])

    plsc.subcore_barrier()

    def chunk(i, carry):
        off = base + i * CH
        pltpu.sync_copy(dst_h.at[pl.ds(off, CH)], idx_v)
        pltpu.sync_copy(m_h.at[pl.ds(off, CH)], rows_v)
        pltpu.sync_copy(rows_v, acc_sh.at[idx_v], add=True)
        return carry

    lax.fori_loop(0, NCHUNK, chunk, 0)
    plsc.subcore_barrier()
    pltpu.sync_copy(acc_sh.at[pl.ds(sid * RPT, RPT)],
                    out_h.at[pl.ds(cid * N + sid * RPT, RPT)])

    @pl.when(sid == NSUB - 1)
    def _():
        pltpu.sync_copy(acc_sh.at[pl.ds(RPT * NSUB, RTAIL)],
                        out_h.at[pl.ds(cid * N + RPT * NSUB, RTAIL)])


def _embed_body(x_ref, w_ref, b_ref, o_ref):
    acc = jnp.dot(x_ref[...], w_ref[...], preferred_element_type=jnp.float32)
    o_ref[...] = jnp.maximum(acc + b_ref[...], 0.0)


def _edgefeat_body(dd_ref, ang_ref, wr_ref, ws_ref, rbf_ref, sbf_ref):
    d = jnp.sqrt(dd_ref[...] + 1e-12)
    e1 = jnp.maximum(1.0 - d * (1.0 / CUT), 0.0)
    e2 = e1 * e1
    env = e2 * e2 * e1
    z = jnp.zeros_like(d)
    rbf = jnp.concatenate(
        [env * jnp.sin((n * math.pi / CUT) * d) for n in range(1, NR + 1)]
        + [z, z], axis=1)
    rbf_ref[...] = jnp.dot(rbf, wr_ref[...],
                           preferred_element_type=jnp.float32)
    sbf_ref[...] = jnp.dot(ang_ref[...], ws_ref[...],
                           preferred_element_type=jnp.float32)


def _msg_body(hs_ref, rbf_ref, sbf_ref, w_ref, b_ref, o_ref):
    pre = hs_ref[...] * rbf_ref[...] + sbf_ref[...]
    acc = jnp.dot(pre, w_ref[...], preferred_element_type=jnp.float32)
    o_ref[...] = jnp.maximum(acc + b_ref[...], 0.0)


def _update_body(h_ref, a0_ref, a1_ref, o_ref):
    o_ref[...] = jnp.maximum(h_ref[...] + a0_ref[...] + a1_ref[...], 0.0)


def _outhead_body(h_ref, wo_ref, wsc_ref, bsc_ref, feat_ref, sc_ref):
    feat = jnp.dot(h_ref[...], wo_ref[...], preferred_element_type=jnp.float32)
    feat_ref[...] = feat
    raw = jnp.sum(feat * wsc_ref[...], axis=1, keepdims=True) + bsc_ref[...]
    sc_ref[...] = jax.nn.sigmoid(raw)


def _head_body(sc_ref, nz_ref, bt_ref, wcls_ref, bcls_ref, feat_hbm,
               nl_ref, pr_ref, sm_ref, row_v, pooled_v, sem):
    scores = sc_ref[...]
    s_sel = scores + nz_ref[...]
    bt = bt_ref[...]
    r_iota = lax.broadcasted_iota(jnp.int32, (NROW, 128), 0)
    c_iota = lax.broadcasted_iota(jnp.int32, (NROW, 128), 1)
    flat = r_iota * 128 + c_iota
    big = jnp.int32(2 ** 30)
    ninf = jnp.float32(-jnp.inf)
    acc = jnp.float32(0.0)
    lane1 = lax.broadcasted_iota(jnp.int32, (1, 128), 1)
    for g in range(B):
        masked = jnp.where(bt == g, s_sel, ninf)
        m1 = jnp.max(masked)
        idx1 = jnp.min(jnp.where(masked == m1, flat, big))
        hit1 = flat == idx1
        masked2 = jnp.where(hit1, ninf, masked)
        m2 = jnp.max(masked2)
        idx2 = jnp.min(jnp.where((masked2 == m2) & (~hit1), flat, big))
        acc = acc + jnp.sum(jnp.where(hit1, scores, 0.0))
        acc = acc + jnp.sum(jnp.where(flat == idx2, scores, 0.0))
        pltpu.make_async_copy(feat_hbm.at[pl.ds(idx1, 1)], row_v, sem).start()
        pltpu.make_async_copy(feat_hbm.at[pl.ds(idx1, 1)], row_v, sem).wait()
        r1 = row_v[...]
        pltpu.make_async_copy(feat_hbm.at[pl.ds(idx2, 1)], row_v, sem).start()
        pltpu.make_async_copy(feat_hbm.at[pl.ds(idx2, 1)], row_v, sem).wait()
        pooled_v[pl.ds(g, 1), :] = 0.5 * (r1 + row_v[...])
        nl_ref[pl.ds(g, 1), :] = jnp.where(
            lane1 == 0, idx1, jnp.where(lane1 == 1, idx2, 0))
    logits = jnp.dot(pooled_v[...], wcls_ref[...],
                     preferred_element_type=jnp.float32) + bcls_ref[...]
    lane = lax.broadcasted_iota(jnp.int32, (B, 128), 1)
    lm = lane < PRED
    mx = jnp.max(jnp.where(lm, logits, ninf), axis=1, keepdims=True)
    ex = jnp.where(lm, jnp.exp(logits - mx), 0.0)
    pr_ref[...] = ex / jnp.sum(ex, axis=1, keepdims=True)
    sm_ref[...] = jnp.reshape(acc / (B * K), (1, 1))


def kernel(x, angle_geom, ca_coords, edge_index, batch,
           W_emb, b_emb, W_rbf, W_sbf,
           W_l0, W_l1, W_l2, b_l0, b_l1, b_l2,
           W_out, W_score, b_score, W_cls, b_cls):
    f32 = jnp.float32
    src = edge_index[0]
    dst = edge_index[1]

    xp = jnp.pad(x, ((0, 0), (0, H - OH)))
    W_emb_p = jnp.pad(W_emb, ((0, H - OH), (0, 0)))
    ang_p = jnp.pad(angle_geom, ((0, 0), (0, H - NS * NR)))
    W_sbf_p = jnp.pad(W_sbf, ((0, H - NS * NR), (0, 0)))
    W_rbf_p = jnp.pad(W_rbf, ((0, 8 - NR), (0, 0)))
    cax = ca_coords[:, 0]
    cay = ca_coords[:, 1]
    caz = ca_coords[:, 2]

    BN = 2000
    BE = 2000

    dd = _dist_sc(cax, cay, caz, src, dst)

    h = pl.pallas_call(
        _embed_body,
        grid=(N // BN,),
        in_specs=[
            pl.BlockSpec((BN, H), lambda i: (i, 0)),
            pl.BlockSpec((H, H), lambda i: (0, 0)),
            pl.BlockSpec((1, H), lambda i: (0, 0)),
        ],
        out_specs=pl.BlockSpec((BN, H), lambda i: (i, 0)),
        out_shape=jax.ShapeDtypeStruct((N, H), f32),
    )(xp, W_emb_p, b_emb.reshape(1, H))

    rbf_e, sbf_e = pl.pallas_call(
        _edgefeat_body,
        grid=(E // BE,),
        in_specs=[
            pl.BlockSpec((BE, 1), lambda i: (i, 0)),
            pl.BlockSpec((BE, H), lambda i: (i, 0)),
            pl.BlockSpec((8, H), lambda i: (0, 0)),
            pl.BlockSpec((H, H), lambda i: (0, 0)),
        ],
        out_specs=[
            pl.BlockSpec((BE, H), lambda i: (i, 0)),
            pl.BlockSpec((BE, H), lambda i: (i, 0)),
        ],
        out_shape=[
            jax.ShapeDtypeStruct((E, H), f32),
            jax.ShapeDtypeStruct((E, H), f32),
        ],
    )(dd.reshape(E, 1), ang_p, W_rbf_p, W_sbf_p)

    zeros_nh = jnp.zeros((N, H), f32)

    for W_l, b_l in ((W_l0, b_l0), (W_l1, b_l1), (W_l2, b_l2)):
        hs = _gather_sc(h, src)
        m = pl.pallas_call(
            _msg_body,
            grid=(E // BE,),
            in_specs=[
                pl.BlockSpec((BE, H), lambda i: (i, 0)),
                pl.BlockSpec((BE, H), lambda i: (i, 0)),
                pl.BlockSpec((BE, H), lambda i: (i, 0)),
                pl.BlockSpec((H, H), lambda i: (0, 0)),
                pl.BlockSpec((1, H), lambda i: (0, 0)),
            ],
            out_specs=pl.BlockSpec((BE, H), lambda i: (i, 0)),
            out_shape=jax.ShapeDtypeStruct((E, H), f32),
        )(hs, rbf_e, sbf_e, W_l, b_l.reshape(1, H))
        agg2 = _scatter_sc(m, dst, zeros_nh)
        h = pl.pallas_call(
            _update_body,
            grid=(N // BN,),
            in_specs=[
                pl.BlockSpec((BN, H), lambda i: (i, 0)),
                pl.BlockSpec((BN, H), lambda i: (i, 0)),
                pl.BlockSpec((BN, H), lambda i: (i, 0)),
            ],
            out_specs=pl.BlockSpec((BN, H), lambda i: (i, 0)),
            out_shape=jax.ShapeDtypeStruct((N, H), f32),
        )(h, agg2[:N], agg2[N:])

    feat, scores2 = pl.pallas_call(
        _outhead_body,
        grid=(N // BN,),
        in_specs=[
            pl.BlockSpec((BN, H), lambda i: (i, 0)),
            pl.BlockSpec((H, HID4), lambda i: (0, 0)),
            pl.BlockSpec((1, HID4), lambda i: (0, 0)),
            pl.BlockSpec((1, 1), lambda i: (0, 0)),
        ],
        out_specs=[
            pl.BlockSpec((BN, HID4), lambda i: (i, 0)),
            pl.BlockSpec((BN, 1), lambda i: (i, 0)),
        ],
        out_shape=[
            jax.ShapeDtypeStruct((N, HID4), f32),
            jax.ShapeDtypeStruct((N, 1), f32),
        ],
    )(h, W_out, W_score.reshape(1, HID4), b_score.reshape(1, 1))
    node_scores = scores2[:, 0]

    noise = 0.01 * jax.random.normal(jax.random.key(42), (N,), dtype=f32)
    sc_p = jnp.pad(node_scores, (0, NP - N)).reshape(NROW, 128)
    nz_p = jnp.pad(noise, (0, NP - N)).reshape(NROW, 128)
    bt_p = jnp.pad(batch, (0, NP - N), constant_values=-1).reshape(NROW, 128)
    W_cls_p = jnp.pad(W_cls, ((0, 0), (0, 128 - PRED)))
    b_cls_p = jnp.pad(b_cls, (0, 128 - PRED)).reshape(1, 128)

    nl_p, pr_p, sm = pl.pallas_call(
        _head_body,
        in_specs=[
            pl.BlockSpec(memory_space=pltpu.VMEM),
            pl.BlockSpec(memory_space=pltpu.VMEM),
            pl.BlockSpec(memory_space=pltpu.VMEM),
            pl.BlockSpec(memory_space=pltpu.VMEM),
            pl.BlockSpec(memory_space=pltpu.VMEM),
            pl.BlockSpec(memory_space=pl.ANY),
        ],
        out_specs=[
            pl.BlockSpec(memory_space=pltpu.VMEM),
            pl.BlockSpec(memory_space=pltpu.VMEM),
            pl.BlockSpec(memory_space=pltpu.VMEM),
        ],
        out_shape=[
            jax.ShapeDtypeStruct((B, 128), jnp.int32),
            jax.ShapeDtypeStruct((B, 128), f32),
            jax.ShapeDtypeStruct((1, 1), f32),
        ],
        scratch_shapes=[
            pltpu.VMEM((1, HID4), f32),
            pltpu.VMEM((B, HID4), f32),
            pltpu.SemaphoreType.DMA,
        ],
    )(sc_p, nz_p, bt_p, W_cls_p, b_cls_p, feat)

    node_list = nl_p[:, :K]
    func_probability = pr_p[:, :PRED]
    score_mean = sm[0, 0]
    return (score_mean, node_scores, node_list, func_probability, feat)


# speedup vs baseline: 1.1907x; 1.1013x over previous
import functools
import math

import jax
import jax.numpy as jnp
from jax import lax
from jax.experimental import pallas as pl
from jax.experimental.pallas import tpu as pltpu
from jax.experimental.pallas import tpu_sc as plsc

N = 10000
E = 320000
H = 128
HID4 = 512
B = 16
K = 2
OH = 23
NR = 6
NS = 7
CUT = 5.0
PRED = 8

NC = 2
NSUB = 16
NW = NC * NSUB
PER_W = E // NW
CH = 80
NCHUNK = PER_W // CH
NBUF = 5
GCH = NBUF * CH
NGRP = PER_W // GCH
RPT = 624
RTAIL = N - RPT * NSUB

_SC_MESH = plsc.VectorSubcoreMesh(core_axis_name="c", subcore_axis_name="s")
_SC_PARAMS = pltpu.CompilerParams(needs_layout_passes=False)

NP = 10240
NROW = NP // 128


@functools.partial(
    pl.kernel,
    mesh=_SC_MESH,
    compiler_params=_SC_PARAMS,
    out_type=jax.ShapeDtypeStruct((E,), jnp.float32),
    scratch_types=[
        pltpu.VMEM((N,), jnp.float32),
        pltpu.VMEM((N,), jnp.float32),
        pltpu.VMEM((N,), jnp.float32),
        pltpu.VMEM((CH,), jnp.int32),
        pltpu.VMEM((CH,), jnp.int32),
        pltpu.VMEM((CH,), jnp.float32),
    ],
)
def _dist_sc(cax_h, cay_h, caz_h, src_h, dst_h, out_h,
             cax_v, cay_v, caz_v, sidx_v, didx_v, dd_v):
    wid = lax.axis_index("s") * NC + lax.axis_index("c")
    base = wid * PER_W
    pltpu.sync_copy(cax_h, cax_v)
    pltpu.sync_copy(cay_h, cay_v)
    pltpu.sync_copy(caz_h, caz_v)

    def chunk(i, carry):
        off = base + i * CH
        pltpu.sync_copy(src_h.at[pl.ds(off, CH)], sidx_v)
        pltpu.sync_copy(dst_h.at[pl.ds(off, CH)], didx_v)
        for j in range(CH // 16):
            si = sidx_v[pl.ds(j * 16, 16)]
            di = didx_v[pl.ds(j * 16, 16)]
            ddx = plsc.load_gather(cax_v, [di]) - plsc.load_gather(cax_v, [si])
            ddy = plsc.load_gather(cay_v, [di]) - plsc.load_gather(cay_v, [si])
            ddz = plsc.load_gather(caz_v, [di]) - plsc.load_gather(caz_v, [si])
            dd_v[pl.ds(j * 16, 16)] = ddx * ddx + ddy * ddy + ddz * ddz
        pltpu.sync_copy(dd_v, out_h.at[pl.ds(off, CH)])
        return carry

    lax.fori_loop(0, NCHUNK, chunk, 0)


@functools.partial(
    pl.kernel,
    mesh=_SC_MESH,
    compiler_params=_SC_PARAMS,
    out_type=jax.ShapeDtypeStruct((E, H), jnp.float32),
    scratch_types=(
        [pltpu.VMEM((CH,), jnp.int32) for _ in range(NBUF)]
        + [
            pltpu.VMEM((2, GCH, H), jnp.float32),
            pltpu.SemaphoreType.DMA,
            pltpu.SemaphoreType.DMA,
            pltpu.SemaphoreType.DMA,
        ]
    ),
)
def _gather_sc(h_h, src_h, out_h, *refs):
    idxr = refs[:NBUF]
    rows_v, sem_i, sem_g, sem_w = refs[NBUF:]
    wid = lax.axis_index("s") * NC + lax.axis_index("c")
    base = wid * PER_W

    def idx_copy(g, b):
        off = base + g * GCH + b * CH
        return pltpu.make_async_copy(src_h.at[pl.ds(off, CH)],
                                     idxr[b], sem_i)

    def wb_copy(g):
        off = base + g * GCH
        return pltpu.make_async_copy(rows_v.at[lax.rem(g, 2)],
                                     out_h.at[pl.ds(off, GCH)], sem_w)

    def gath(g, b):
        p = lax.rem(g, 2)
        return pltpu.make_async_copy(
            h_h.at[idxr[b]],
            rows_v.at[p, pl.ds(b * CH, CH), :], sem_g)

    for b in range(NBUF):
        idx_copy(0, b).start()

    def group(g, carry):
        @pl.when(g >= 2)
        def _():
            wb_copy(g - 2).wait()

        for b in range(NBUF):
            idx_copy(g, b).wait()
        for b in range(NBUF):
            gath(g, b).start()
        for b in range(NBUF):
            gath(g, b).wait()

        @pl.when(g + 1 < NGRP)
        def _():
            for b in range(NBUF):
                idx_copy(g + 1, b).start()

        wb_copy(g).start()
        return carry

    lax.fori_loop(0, NGRP, group, 0)
    wb_copy(NGRP - 2).wait()
    wb_copy(NGRP - 1).wait()


@functools.partial(
    pl.kernel,
    mesh=_SC_MESH,
    compiler_params=_SC_PARAMS,
    out_type=jax.ShapeDtypeStruct((NC * N, H), jnp.float32),
    scratch_types=[
        pltpu.VMEM((CH,), jnp.int32),
        pltpu.VMEM((CH,), jnp.int32),
        pltpu.VMEM((2, CH, H), jnp.float32),
        pltpu.VMEM_SHARED((N, H), jnp.float32),
        pltpu.SemaphoreType.DMA,
        pltpu.SemaphoreType.DMA,
        pltpu.SemaphoreType.DMA,
    ],
)
def _scatter_sc(m_h, dst_h, zeros_h, out_h, idx0, idx1, rows_v, acc_sh,
                sem_i, sem_r, sem_s):
    idxr = (idx0, idx1)
    cid = lax.axis_index("c")
    sid = lax.axis_index("s")
    wid = sid * NC + cid
    base = wid * PER_W

    def idx_start(g, pp):
        off = base + g * CH
        pltpu.make_async_copy(dst_h.at[pl.ds(off, CH)],
                              idxr[pp], sem_i).start()

    def rows_copy(g):
        off = base + g * CH
        return pltpu.make_async_copy(m_h.at[pl.ds(off, CH)],
                                     rows_v.at[lax.rem(g, 2)], sem_r)

    def scat_start(g, pp):
        pltpu.async_copy(rows_v.at[lax.rem(g, 2)],
                         acc_sh.at[idxr[pp]], sem_s, add=True)

    def idx_wait():
        pltpu.make_async_copy(dst_h.at[pl.ds(base, CH)],
                              idxr[0], sem_i).wait()

    def scat_wait():
        pltpu.make_async_copy(rows_v.at[0],
                              acc_sh.at[idxr[0]], sem_s).wait()

    idx_start(0, 0)
    rows_copy(0).start()

    pltpu.sync_copy(zeros_h.at[pl.ds(sid * RPT, RPT)],
                    acc_sh.at[pl.ds(sid * RPT, RPT)])

    @pl.when(sid == NSUB - 1)
    def _():
        pltpu.sync_copy(zeros_h.at[pl.ds(RPT * NSUB, RTAIL)],
                        acc_sh.at[pl.ds(RPT * NSUB, RTAIL)])

    plsc.subcore_barrier()

    def group(g, carry):
        p_is_0 = lax.rem(g, 2) == 0
        idx_wait()
        rows_copy(g).wait()

        @pl.when(g >= 1)
        def _():
            scat_wait()

        @pl.when(g + 1 < NCHUNK)
        def _():
            @pl.when(p_is_0)
            def _():
                idx_start(g + 1, 1)

            @pl.when(jnp.logical_not(p_is_0))
            def _():
                idx_start(g + 1, 0)

            rows_copy(g + 1).start()

        @pl.when(p_is_0)
        def _():
            scat_start(g, 0)

        @pl.when(jnp.logical_not(p_is_0))
        def _():
            scat_start(g, 1)

        return carry

    lax.fori_loop(0, NCHUNK, group, 0)
    scat_wait()
    plsc.subcore_barrier()
    pltpu.sync_copy(acc_sh.at[pl.ds(sid * RPT, RPT)],
                    out_h.at[pl.ds(cid * N + sid * RPT, RPT)])

    @pl.when(sid == NSUB - 1)
    def _():
        pltpu.sync_copy(acc_sh.at[pl.ds(RPT * NSUB, RTAIL)],
                        out_h.at[pl.ds(cid * N + RPT * NSUB, RTAIL)])


def _embed_body(x_ref, w_ref, b_ref, o_ref):
    acc = jnp.dot(x_ref[...], w_ref[...], preferred_element_type=jnp.float32)
    o_ref[...] = jnp.maximum(acc + b_ref[...], 0.0)


def _edgefeat_body(dd_ref, ang_ref, wr_ref, ws_ref, rbf_ref, sbf_ref):
    d = jnp.sqrt(dd_ref[...] + 1e-12)
    e1 = jnp.maximum(1.0 - d * (1.0 / CUT), 0.0)
    e2 = e1 * e1
    env = e2 * e2 * e1
    z = jnp.zeros_like(d)
    rbf = jnp.concatenate(
        [env * jnp.sin((n * math.pi / CUT) * d) for n in range(1, NR + 1)]
        + [z, z], axis=1)
    rbf_ref[...] = jnp.dot(rbf, wr_ref[...],
                           preferred_element_type=jnp.float32)
    sbf_ref[...] = jnp.dot(ang_ref[...], ws_ref[...],
                           preferred_element_type=jnp.float32)


def _msg_body(hs_ref, rbf_ref, sbf_ref, w_ref, b_ref, o_ref):
    pre = hs_ref[...] * rbf_ref[...] + sbf_ref[...]
    acc = jnp.dot(pre, w_ref[...], preferred_element_type=jnp.float32)
    o_ref[...] = jnp.maximum(acc + b_ref[...], 0.0)


def _update_body(h_ref, a0_ref, a1_ref, o_ref):
    o_ref[...] = jnp.maximum(h_ref[...] + a0_ref[...] + a1_ref[...], 0.0)


def _outhead_body(h_ref, wo_ref, wsc_ref, bsc_ref, feat_ref, sc_ref):
    feat = jnp.dot(h_ref[...], wo_ref[...], preferred_element_type=jnp.float32)
    feat_ref[...] = feat
    raw = jnp.sum(feat * wsc_ref[...], axis=1, keepdims=True) + bsc_ref[...]
    sc_ref[...] = jax.nn.sigmoid(raw)


def _head_body(sc_ref, nz_ref, bt_ref, wcls_ref, bcls_ref, feat_hbm,
               nl_ref, pr_ref, sm_ref, row_v, pooled_v, sem):
    scores = sc_ref[...]
    s_sel = scores + nz_ref[...]
    bt = bt_ref[...]
    r_iota = lax.broadcasted_iota(jnp.int32, (NROW, 128), 0)
    c_iota = lax.broadcasted_iota(jnp.int32, (NROW, 128), 1)
    flat = r_iota * 128 + c_iota
    big = jnp.int32(2 ** 30)
    ninf = jnp.float32(-jnp.inf)
    acc = jnp.float32(0.0)
    lane1 = lax.broadcasted_iota(jnp.int32, (1, 128), 1)
    for g in range(B):
        masked = jnp.where(bt == g, s_sel, ninf)
        m1 = jnp.max(masked)
        idx1 = jnp.min(jnp.where(masked == m1, flat, big))
        hit1 = flat == idx1
        masked2 = jnp.where(hit1, ninf, masked)
        m2 = jnp.max(masked2)
        idx2 = jnp.min(jnp.where((masked2 == m2) & (~hit1), flat, big))
        acc = acc + jnp.sum(jnp.where(hit1, scores, 0.0))
        acc = acc + jnp.sum(jnp.where(flat == idx2, scores, 0.0))
        pltpu.make_async_copy(feat_hbm.at[pl.ds(idx1, 1)], row_v, sem).start()
        pltpu.make_async_copy(feat_hbm.at[pl.ds(idx1, 1)], row_v, sem).wait()
        r1 = row_v[...]
        pltpu.make_async_copy(feat_hbm.at[pl.ds(idx2, 1)], row_v, sem).start()
        pltpu.make_async_copy(feat_hbm.at[pl.ds(idx2, 1)], row_v, sem).wait()
        pooled_v[pl.ds(g, 1), :] = 0.5 * (r1 + row_v[...])
        nl_ref[pl.ds(g, 1), :] = jnp.where(
            lane1 == 0, idx1, jnp.where(lane1 == 1, idx2, 0))
    logits = jnp.dot(pooled_v[...], wcls_ref[...],
                     preferred_element_type=jnp.float32) + bcls_ref[...]
    lane = lax.broadcasted_iota(jnp.int32, (B, 128), 1)
    lm = lane < PRED
    mx = jnp.max(jnp.where(lm, logits, ninf), axis=1, keepdims=True)
    ex = jnp.where(lm, jnp.exp(logits - mx), 0.0)
    pr_ref[...] = ex / jnp.sum(ex, axis=1, keepdims=True)
    sm_ref[...] = jnp.reshape(acc / (B * K), (1, 1))


def kernel(x, angle_geom, ca_coords, edge_index, batch,
           W_emb, b_emb, W_rbf, W_sbf,
           W_l0, W_l1, W_l2, b_l0, b_l1, b_l2,
           W_out, W_score, b_score, W_cls, b_cls):
    f32 = jnp.float32
    src = edge_index[0]
    dst = edge_index[1]

    xp = jnp.pad(x, ((0, 0), (0, H - OH)))
    W_emb_p = jnp.pad(W_emb, ((0, H - OH), (0, 0)))
    ang_p = jnp.pad(angle_geom, ((0, 0), (0, H - NS * NR)))
    W_sbf_p = jnp.pad(W_sbf, ((0, H - NS * NR), (0, 0)))
    W_rbf_p = jnp.pad(W_rbf, ((0, 8 - NR), (0, 0)))
    cax = ca_coords[:, 0]
    cay = ca_coords[:, 1]
    caz = ca_coords[:, 2]

    BN = 2000
    BE = 2000

    dd = _dist_sc(cax, cay, caz, src, dst)

    h = pl.pallas_call(
        _embed_body,
        grid=(N // BN,),
        in_specs=[
            pl.BlockSpec((BN, H), lambda i: (i, 0)),
            pl.BlockSpec((H, H), lambda i: (0, 0)),
            pl.BlockSpec((1, H), lambda i: (0, 0)),
        ],
        out_specs=pl.BlockSpec((BN, H), lambda i: (i, 0)),
        out_shape=jax.ShapeDtypeStruct((N, H), f32),
    )(xp, W_emb_p, b_emb.reshape(1, H))

    rbf_e, sbf_e = pl.pallas_call(
        _edgefeat_body,
        grid=(E // BE,),
        in_specs=[
            pl.BlockSpec((BE, 1), lambda i: (i, 0)),
            pl.BlockSpec((BE, H), lambda i: (i, 0)),
            pl.BlockSpec((8, H), lambda i: (0, 0)),
            pl.BlockSpec((H, H), lambda i: (0, 0)),
        ],
        out_specs=[
            pl.BlockSpec((BE, H), lambda i: (i, 0)),
            pl.BlockSpec((BE, H), lambda i: (i, 0)),
        ],
        out_shape=[
            jax.ShapeDtypeStruct((E, H), f32),
            jax.ShapeDtypeStruct((E, H), f32),
        ],
    )(dd.reshape(E, 1), ang_p, W_rbf_p, W_sbf_p)

    zeros_nh = jnp.zeros((N, H), f32)

    for W_l, b_l in ((W_l0, b_l0), (W_l1, b_l1), (W_l2, b_l2)):
        hs = _gather_sc(h, src)
        m = pl.pallas_call(
            _msg_body,
            grid=(E // BE,),
            in_specs=[
                pl.BlockSpec((BE, H), lambda i: (i, 0)),
                pl.BlockSpec((BE, H), lambda i: (i, 0)),
                pl.BlockSpec((BE, H), lambda i: (i, 0)),
                pl.BlockSpec((H, H), lambda i: (0, 0)),
                pl.BlockSpec((1, H), lambda i: (0, 0)),
            ],
            out_specs=pl.BlockSpec((BE, H), lambda i: (i, 0)),
            out_shape=jax.ShapeDtypeStruct((E, H), f32),
        )(hs, rbf_e, sbf_e, W_l, b_l.reshape(1, H))
        agg2 = _scatter_sc(m, dst, zeros_nh)
        h = pl.pallas_call(
            _update_body,
            grid=(N // BN,),
            in_specs=[
                pl.BlockSpec((BN, H), lambda i: (i, 0)),
                pl.BlockSpec((BN, H), lambda i: (i, 0)),
                pl.BlockSpec((BN, H), lambda i: (i, 0)),
            ],
            out_specs=pl.BlockSpec((BN, H), lambda i: (i, 0)),
            out_shape=jax.ShapeDtypeStruct((N, H), f32),
        )(h, agg2[:N], agg2[N:])

    feat, scores2 = pl.pallas_call(
        _outhead_body,
        grid=(N // BN,),
        in_specs=[
            pl.BlockSpec((BN, H), lambda i: (i, 0)),
            pl.BlockSpec((H, HID4), lambda i: (0, 0)),
            pl.BlockSpec((1, HID4), lambda i: (0, 0)),
            pl.BlockSpec((1, 1), lambda i: (0, 0)),
        ],
        out_specs=[
            pl.BlockSpec((BN, HID4), lambda i: (i, 0)),
            pl.BlockSpec((BN, 1), lambda i: (i, 0)),
        ],
        out_shape=[
            jax.ShapeDtypeStruct((N, HID4), f32),
            jax.ShapeDtypeStruct((N, 1), f32),
        ],
    )(h, W_out, W_score.reshape(1, HID4), b_score.reshape(1, 1))
    node_scores = scores2[:, 0]

    noise = 0.01 * jax.random.normal(jax.random.key(42), (N,), dtype=f32)
    sc_p = jnp.pad(node_scores, (0, NP - N)).reshape(NROW, 128)
    nz_p = jnp.pad(noise, (0, NP - N)).reshape(NROW, 128)
    bt_p = jnp.pad(batch, (0, NP - N), constant_values=-1).reshape(NROW, 128)
    W_cls_p = jnp.pad(W_cls, ((0, 0), (0, 128 - PRED)))
    b_cls_p = jnp.pad(b_cls, (0, 128 - PRED)).reshape(1, 128)

    nl_p, pr_p, sm = pl.pallas_call(
        _head_body,
        in_specs=[
            pl.BlockSpec(memory_space=pltpu.VMEM),
            pl.BlockSpec(memory_space=pltpu.VMEM),
            pl.BlockSpec(memory_space=pltpu.VMEM),
            pl.BlockSpec(memory_space=pltpu.VMEM),
            pl.BlockSpec(memory_space=pltpu.VMEM),
            pl.BlockSpec(memory_space=pl.ANY),
        ],
        out_specs=[
            pl.BlockSpec(memory_space=pltpu.VMEM),
            pl.BlockSpec(memory_space=pltpu.VMEM),
            pl.BlockSpec(memory_space=pltpu.VMEM),
        ],
        out_shape=[
            jax.ShapeDtypeStruct((B, 128), jnp.int32),
            jax.ShapeDtypeStruct((B, 128), f32),
            jax.ShapeDtypeStruct((1, 1), f32),
        ],
        scratch_shapes=[
            pltpu.VMEM((1, HID4), f32),
            pltpu.VMEM((B, HID4), f32),
            pltpu.SemaphoreType.DMA,
        ],
    )(sc_p, nz_p, bt_p, W_cls_p, b_cls_p, feat)

    node_list = nl_p[:, :K]
    func_probability = pr_p[:, :PRED]
    score_mean = sm[0, 0]
    return (score_mean, node_scores, node_list, func_probability, feat)
